# Initial kernel scaffold; baseline (speedup 1.0000x reference)
import jax
import jax.numpy as jnp
from jax.experimental import pallas as pl


def kernel(x, W_gcn, alpha, w1, r1, b1, w2, r2, b2, w3, r3, b3, edge_index):
    raise NotImplementedError("write your pallas kernel here")


# speedup vs baseline: 6.2045x; 6.2045x over previous
import functools

import jax
import jax.numpy as jnp
from jax import lax
from jax.experimental import pallas as pl
from jax.experimental.pallas import tpu as pltpu
from jax.experimental.pallas import tpu_sc as plsc

N = 10000
E = 320000
D = 128

NC = 2
NS = 16
NW = NC * NS
CHUNK = 128
EPAD = ((E + NW * CHUNK - 1) // (NW * CHUNK)) * (NW * CHUNK)
EPT = EPAD // NW
NCHUNK = EPT // CHUNK
NPAD = 10240
ZROWS = 64
ROWS_PT = NPAD // NS

_MESH = plsc.VectorSubcoreMesh(
    core_axis_name="c", subcore_axis_name="s", num_cores=NC, num_subcores=NS
)


def _zero_fill(zbuf, width):
    def body(i, _):
        for j in range(width // 16):
            zbuf[i, pl.ds(j * 16, 16)] = jnp.zeros((16,), jnp.float32)
        return 0
    lax.fori_loop(0, ZROWS, body, 0, unroll=False)


def _spmm_body(g_hbm, col_hbm, row_hbm, out_hbm, colv, rowv, buf, acc, zbuf, sem):
    c = lax.axis_index("c")
    s = lax.axis_index("s")
    wid = s * NC + c

    _zero_fill(zbuf, D)
    zb = s * ROWS_PT
    def zacc(k, _):
        pltpu.sync_copy(zbuf, acc.at[pl.ds(zb + k * ZROWS, ZROWS)])
        return 0
    lax.fori_loop(0, ROWS_PT // ZROWS, zacc, 0, unroll=False)
    plsc.subcore_barrier()

    ebase = wid * EPT
    def step(j, _):
        off = ebase + j * CHUNK
        pltpu.sync_copy(col_hbm.at[pl.ds(off, CHUNK)], colv)
        pltpu.sync_copy(row_hbm.at[pl.ds(off, CHUNK)], rowv)
        pltpu.async_copy(g_hbm.at[colv], buf, sem).wait()
        pltpu.sync_copy(buf, acc.at[rowv], add=True)
        return 0
    lax.fori_loop(0, NCHUNK, step, 0, unroll=False)
    plsc.subcore_barrier()

    ob = s * ROWS_PT
    pltpu.sync_copy(acc.at[pl.ds(ob, ROWS_PT)],
                    out_hbm.at[pl.ds(c * NPAD + ob, ROWS_PT)])


_spmm_call = pl.kernel(
    _spmm_body,
    out_type=jax.ShapeDtypeStruct((NC * NPAD, D), jnp.float32),
    mesh=_MESH,
    scratch_types=[
        pltpu.VMEM((CHUNK,), jnp.int32),
        pltpu.VMEM((CHUNK,), jnp.int32),
        pltpu.VMEM((CHUNK, D), jnp.float32),
        pltpu.VMEM_SHARED((NPAD, D), jnp.float32),
        pltpu.VMEM((ZROWS, D), jnp.float32),
        pltpu.SemaphoreType.DMA,
    ],
)


def _deg_body(row_hbm, out_hbm, rowv, ones, acc, zbuf):
    c = lax.axis_index("c")
    s = lax.axis_index("s")
    wid = s * NC + c

    _zero_fill(zbuf, D)
    def orow(i, _):
        for j in range(D // 16):
            ones[i, pl.ds(j * 16, 16)] = jnp.ones((16,), jnp.float32)
        return 0
    lax.fori_loop(0, CHUNK, orow, 0, unroll=False)

    zb = s * ROWS_PT
    def zacc(k, _):
        pltpu.sync_copy(zbuf, acc.at[pl.ds(zb + k * ZROWS, ZROWS)])
        return 0
    lax.fori_loop(0, ROWS_PT // ZROWS, zacc, 0, unroll=False)
    plsc.subcore_barrier()

    ebase = wid * EPT
    def step(j, _):
        off = ebase + j * CHUNK
        pltpu.sync_copy(row_hbm.at[pl.ds(off, CHUNK)], rowv)
        pltpu.sync_copy(ones, acc.at[rowv], add=True)
        return 0
    lax.fori_loop(0, NCHUNK, step, 0, unroll=False)
    plsc.subcore_barrier()

    ob = s * ROWS_PT
    pltpu.sync_copy(acc.at[pl.ds(ob, ROWS_PT)],
                    out_hbm.at[pl.ds(c * NPAD + ob, ROWS_PT)])


_deg_call = pl.kernel(
    _deg_body,
    out_type=jax.ShapeDtypeStruct((NC * NPAD, D), jnp.float32),
    mesh=_MESH,
    scratch_types=[
        pltpu.VMEM((CHUNK,), jnp.int32),
        pltpu.VMEM((CHUNK, D), jnp.float32),
        pltpu.VMEM_SHARED((NPAD, D), jnp.float32),
        pltpu.VMEM((ZROWS, D), jnp.float32),
    ],
)


BLK = 1000
GRID = N // BLK

_row_spec = pl.BlockSpec((BLK, D), lambda i: (i, 0))
_col_spec = pl.BlockSpec((BLK, 1), lambda i: (i, 0))
_w_spec = pl.BlockSpec((D, D), lambda i: (0, 0))
_b_spec = pl.BlockSpec((1, D), lambda i: (0, 0))
_a_spec = pl.BlockSpec((1, 1), lambda i: (0, 0))


def _pre_body(dp0, dp1, x, dis_o, inv_o, g0_o):
    deg = dp0[:, 0:1] + dp1[:, 0:1]
    pos = deg > 0.5
    dsafe = jnp.maximum(deg, 1.0)
    dis = jnp.where(pos, lax.rsqrt(dsafe), 0.0)
    dis_o[...] = dis
    inv_o[...] = dis / dsafe
    g0_o[...] = x[...] * dis


_pre_call = pl.pallas_call(
    _pre_body,
    grid=(GRID,),
    in_specs=[_row_spec, _row_spec, _row_spec],
    out_specs=[_col_spec, _col_spec, _row_spec],
    out_shape=[
        jax.ShapeDtypeStruct((N, 1), jnp.float32),
        jax.ShapeDtypeStruct((N, 1), jnp.float32),
        jax.ShapeDtypeStruct((N, D), jnp.float32),
    ],
)


def _gcn_body(p0, p1, x, dis, wg, alpha, h_o, g_o):
    agg = (p0[...] + p1[...]) * dis[...]
    h = x[...] + alpha[0, 0] * jnp.dot(agg, wg[...], preferred_element_type=jnp.float32)
    h_o[...] = h
    g_o[...] = h * dis[...]


_gcn_call = pl.pallas_call(
    _gcn_body,
    grid=(GRID,),
    in_specs=[_row_spec, _row_spec, _row_spec, _col_spec, _w_spec, _a_spec],
    out_specs=[_row_spec, _row_spec],
    out_shape=[
        jax.ShapeDtypeStruct((N, D), jnp.float32),
        jax.ShapeDtypeStruct((N, D), jnp.float32),
    ],
)


def _sage_body(p0, p1, h_prev, dis, inv, w, r, b, h_o, g_o=None):
    mean = (p0[...] + p1[...]) * inv[...]
    o = (jnp.dot(mean, w[...], preferred_element_type=jnp.float32)
         + jnp.dot(h_prev[...], r[...], preferred_element_type=jnp.float32)
         + b[...])
    if g_o is not None:
        o = jnp.maximum(o, 0.0)
        g_o[...] = o * dis[...]
    h_o[...] = o


_sage_specs = [_row_spec, _row_spec, _row_spec, _col_spec, _col_spec,
               _w_spec, _w_spec, _b_spec]

_sage_relu_call = pl.pallas_call(
    _sage_body,
    grid=(GRID,),
    in_specs=_sage_specs,
    out_specs=[_row_spec, _row_spec],
    out_shape=[
        jax.ShapeDtypeStruct((N, D), jnp.float32),
        jax.ShapeDtypeStruct((N, D), jnp.float32),
    ],
)

_sage_final_call = pl.pallas_call(
    functools.partial(_sage_body, g_o=None),
    grid=(GRID,),
    in_specs=_sage_specs,
    out_specs=_row_spec,
    out_shape=jax.ShapeDtypeStruct((N, D), jnp.float32),
)


def kernel(x, W_gcn, alpha, w1, r1, b1, w2, r2, b2, w3, r3, b3, edge_index):
    row = edge_index[0]
    col = edge_index[1]
    pad = EPAD - E
    rowp = jnp.concatenate([row, jnp.full((pad,), N, jnp.int32)])
    colp = jnp.concatenate([col, jnp.zeros((pad,), jnp.int32)])

    dpart = _deg_call(rowp)
    dis, inv, g0 = _pre_call(dpart[:N], dpart[NPAD:NPAD + N], x)

    p = _spmm_call(g0, colp, rowp)
    h0, g1 = _gcn_call(p[:N], p[NPAD:NPAD + N], x, dis, W_gcn, alpha.reshape(1, 1))

    p = _spmm_call(g1, colp, rowp)
    h1, g2 = _sage_relu_call(p[:N], p[NPAD:NPAD + N], h0, dis, inv,
                             w1, r1, b1.reshape(1, D))

    p = _spmm_call(g2, colp, rowp)
    h2, g3 = _sage_relu_call(p[:N], p[NPAD:NPAD + N], h1, dis, inv,
                             w2, r2, b2.reshape(1, D))

    p = _spmm_call(g3, colp, rowp)
    return _sage_final_call(p[:N], p[NPAD:NPAD + N], h2, dis, inv,
                            w3, r3, b3.reshape(1, D))
